# baseline (speedup 1.0000x reference)
import functools

import jax
import jax.numpy as jnp
from jax import lax
from jax.experimental import pallas as pl
from jax.experimental.pallas import tpu as pltpu
from jax.experimental.pallas import tpu_sc as plsc

B = 64
N = 256
DIM = 192
H = 6
HD = DIM // H
TOPK = 64
TABLE = (2 * 16 - 1) * (2 * 16 - 1)
SCALE = HD ** -0.5


_NW = 32
_TABPAD = 5792


def _rpb_body(table_hbm, rpi_hbm, rpb_hbm, tab_v, idx_v, out_v):
    npw = (N * N) // _NW
    wid = lax.axis_index("s") * 2 + lax.axis_index("c")
    base = wid * npw
    pltpu.sync_copy(table_hbm, tab_v)
    pltpu.sync_copy(rpi_hbm.at[pl.ds(base, npw)], idx_v)

    def jbody(j, carry):
        off = pl.multiple_of(16 * j, 16)
        idx = idx_v[pl.ds(off, 16)]
        for h in range(H):
            out_v[h, pl.ds(off, 16)] = plsc.load_gather(
                tab_v, [idx + (h * TABLE)])
        return carry

    lax.fori_loop(0, npw // 16, jbody, 0)
    for h in range(H):
        pltpu.sync_copy(out_v.at[h], rpb_hbm.at[h, pl.ds(base, npw)])


def _rpb_stage(table_t, rpi_flat):
    npw = (N * N) // _NW
    mesh = plsc.VectorSubcoreMesh(core_axis_name="c", subcore_axis_name="s")
    f = pl.kernel(
        _rpb_body,
        out_type=jax.ShapeDtypeStruct((H, N * N), jnp.float32),
        mesh=mesh,
        scratch_types=[
            pltpu.VMEM((_TABPAD,), jnp.float32),
            pltpu.VMEM((npw,), jnp.int32),
            pltpu.VMEM((H, npw), jnp.float32),
        ],
        compiler_params=pltpu.CompilerParams(
            use_tc_tiling_on_sc=False, needs_layout_passes=False),
    )
    return f(table_t, rpi_flat)


def _attn_body(x_ref, rpb_ref, attn_ref):
    x = x_ref[0]
    for h in range(H):
        q = x[:, h * HD:(h + 1) * HD] * SCALE
        k = x[:, DIM + h * HD:DIM + (h + 1) * HD]
        s = lax.dot_general(q, k, (((1,), (1,)), ((), ())),
                            preferred_element_type=jnp.float32)
        s = s + rpb_ref[h]
        m = jnp.max(s, axis=-1, keepdims=True)
        e = jnp.exp(s - m)
        attn_ref[0, h] = e / jnp.sum(e, axis=-1, keepdims=True)


def _attn_stage(qkvp, rpb):
    return pl.pallas_call(
        _attn_body,
        grid=(B,),
        in_specs=[
            pl.BlockSpec((1, N, 4 * DIM), lambda b: (b, 0, 0)),
            pl.BlockSpec((H, N, N), lambda b: (0, 0, 0)),
        ],
        out_specs=pl.BlockSpec((1, H, N, N), lambda b: (b, 0, 0, 0)),
        out_shape=jax.ShapeDtypeStruct((B, H, N, N), jnp.float32),
    )(qkvp, rpb)


_R = B * H * N
_RPW = _R // _NW
_CHUNK = 32
_NCHUNK = _RPW // _CHUNK


def _vsortd(kv):
    return plsc.sort_key_val(kv[0], kv[1], descending=True)


def _vrev(kv):
    return lax.rev(kv[0], (0,)), lax.rev(kv[1], (0,))


def _cleaner(a, b):
    ak, av = a
    bk, bv = b
    c = ak >= bk
    hi = (jnp.where(c, ak, bk), jnp.where(c, av, bv))
    lo = (jnp.where(c, bk, ak), jnp.where(c, bv, av))
    return hi, lo


def _cleaner_hi(a, b):
    ak, av = a
    bk, bv = b
    c = ak >= bk
    return jnp.where(c, ak, bk), jnp.where(c, av, bv)


def _merge2(a, b):
    hi, lo = _cleaner(a, _vrev(b))
    return [_vsortd(hi), _vsortd(lo)]


def _merge4(a, b):
    rb = [_vrev(b[1]), _vrev(b[0])]
    h0, l0 = _cleaner(a[0], rb[0])
    h1, l1 = _cleaner(a[1], rb[1])
    th, tl = _cleaner(h0, h1)
    bh, bl = _cleaner(l0, l1)
    return [_vsortd(th), _vsortd(tl), _vsortd(bh), _vsortd(bl)]


def _merge8_top(a, b):
    rb = [_vrev(b[3]), _vrev(b[2]), _vrev(b[1]), _vrev(b[0])]
    hi = [_cleaner_hi(a[i], rb[i]) for i in range(4)]
    a0, a2 = _cleaner(hi[0], hi[2])
    a1, a3 = _cleaner(hi[1], hi[3])
    b0, b1 = _cleaner(a0, a1)
    b2, b3 = _cleaner(a2, a3)
    return [_vsortd(b0), _vsortd(b1), _vsortd(b2), _vsortd(b3)]


def _row_top64(load16):
    runs16 = [load16(j) for j in range(16)]
    runs32 = [_merge2(runs16[2 * i], runs16[2 * i + 1]) for i in range(8)]
    runs64 = [_merge4(runs32[2 * i], runs32[2 * i + 1]) for i in range(4)]
    t0 = _merge8_top(runs64[0], runs64[1])
    t1 = _merge8_top(runs64[2], runs64[3])
    return _merge8_top(t0, t1)


def _topk_body(attn_hbm, tv_hbm, ti_hbm, bufs, tvb, tib, isems, osems):
    wid = lax.axis_index("s") * 2 + lax.axis_index("c")
    base = wid * _RPW
    iota = lax.iota(jnp.int32, 16)
    idxc = [iota + 16 * j for j in range(16)]

    def in_copy(c, par):
        return pltpu.make_async_copy(
            attn_hbm.at[pl.ds(base + c * _CHUNK, _CHUNK)], bufs.at[par],
            isems.at[par])

    def out_copy_v(c, par):
        return pltpu.make_async_copy(
            tvb.at[par], tv_hbm.at[pl.ds(base + c * _CHUNK, _CHUNK)],
            osems.at[par])

    def out_copy_i(c, par):
        return pltpu.make_async_copy(
            tib.at[par], ti_hbm.at[pl.ds(base + c * _CHUNK, _CHUNK)],
            osems.at[par])

    in_copy(0, 0).start()
    in_copy(1, 1).start()

    def pair_body(c2, carry):
        for par in range(2):
            c = 2 * c2 + par
            in_copy(c, par).wait()

            @pl.when(c >= 2)
            def _():
                out_copy_v(c - 2, par).wait()
                out_copy_i(c - 2, par).wait()

            @plsc.parallel_loop(0, _CHUNK, unroll=2)
            def row_body(i):
                def load16(j):
                    return _vsortd((bufs[par, i, pl.ds(16 * j, 16)], idxc[j]))

                t = _row_top64(load16)
                for k in range(4):
                    tvb[par, i, pl.ds(16 * k, 16)] = t[k][0]
                    tib[par, i, pl.ds(16 * k, 16)] = t[k][1]

            @pl.when(c + 2 < _NCHUNK)
            def _():
                in_copy(c + 2, par).start()

            out_copy_v(c, par).start()
            out_copy_i(c, par).start()
        return carry

    lax.fori_loop(0, _NCHUNK // 2, pair_body, 0)
    for par in range(2):
        out_copy_v(_NCHUNK - 2 + par, par).wait()
        out_copy_i(_NCHUNK - 2 + par, par).wait()


def _topk_stage(attn_rows):
    mesh = plsc.VectorSubcoreMesh(core_axis_name="c", subcore_axis_name="s")
    f = pl.kernel(
        _topk_body,
        out_type=(
            jax.ShapeDtypeStruct((_R, TOPK), jnp.float32),
            jax.ShapeDtypeStruct((_R, TOPK), jnp.int32),
        ),
        mesh=mesh,
        scratch_types=[
            pltpu.VMEM((2, _CHUNK, N), jnp.float32),
            pltpu.VMEM((2, _CHUNK, TOPK), jnp.float32),
            pltpu.VMEM((2, _CHUNK, TOPK), jnp.int32),
            pltpu.SemaphoreType.DMA((2,)),
            pltpu.SemaphoreType.DMA((2,)),
        ],
        compiler_params=pltpu.CompilerParams(
            use_tc_tiling_on_sc=False, needs_layout_passes=False),
    )
    return f(attn_rows)


def _out_body(attn_ref, tv_ref, x_ref, w_ref, bias_ref, out_ref):
    x = x_ref[0]
    parts = []
    for h in range(H):
        a = attn_ref[0, h]
        thr = tv_ref[0, h][:, TOPK - 1:TOPK]
        am = jnp.where(a >= thr, a, 0.0)
        v = x[:, 2 * DIM + h * HD:2 * DIM + (h + 1) * HD]
        lepe = x[:, 3 * DIM + h * HD:3 * DIM + (h + 1) * HD]
        parts.append(lax.dot_general(am, v, (((1,), (0,)), ((), ())),
                                     preferred_element_type=jnp.float32) + lepe)
    xo = jnp.concatenate(parts, axis=-1)
    out = lax.dot_general(xo, w_ref[...], (((1,), (1,)), ((), ())),
                          preferred_element_type=jnp.float32)
    out_ref[0] = out + bias_ref[...]


def _out_stage(attn, topv, qkvp, proj_w, proj_b):
    return pl.pallas_call(
        _out_body,
        grid=(B,),
        in_specs=[
            pl.BlockSpec((1, H, N, N), lambda b: (b, 0, 0, 0)),
            pl.BlockSpec((1, H, N, TOPK), lambda b: (b, 0, 0, 0)),
            pl.BlockSpec((1, N, 4 * DIM), lambda b: (b, 0, 0)),
            pl.BlockSpec((DIM, DIM), lambda b: (0, 0)),
            pl.BlockSpec((1, DIM), lambda b: (0, 0)),
        ],
        out_specs=pl.BlockSpec((1, N, DIM), lambda b: (b, 0, 0)),
        out_shape=jax.ShapeDtypeStruct((B, N, DIM), jnp.float32),
    )(attn, topv, qkvp, proj_w, proj_b)


def kernel(qkvp, table, proj_w, proj_b, rpi):
    table_flat = jnp.pad(table.T.reshape(-1), (0, _TABPAD - H * TABLE))
    rpb = _rpb_stage(table_flat, rpi.reshape(-1).astype(jnp.int32))
    rpb = rpb.reshape(H, N, N)

    attn = _attn_stage(qkvp, rpb)

    tv, ti = _topk_stage(attn.reshape(_R, N))
    topv = tv.reshape(B, H, N, TOPK)
    topi = ti.reshape(B, H, N, TOPK)

    xo = _out_stage(attn, topv, qkvp, proj_w, proj_b.reshape(1, DIM))
    return xo, topv, topi


# speedup vs baseline: 1.5090x; 1.5090x over previous
import functools

import jax
import jax.numpy as jnp
from jax import lax
from jax.experimental import pallas as pl
from jax.experimental.pallas import tpu as pltpu
from jax.experimental.pallas import tpu_sc as plsc

B = 64
N = 256
DIM = 192
H = 6
HD = DIM // H
TOPK = 64
TABLE = (2 * 16 - 1) * (2 * 16 - 1)
SCALE = HD ** -0.5


_NW = 32
_TABPAD = 5792


def _rpb_body(table_hbm, rpi_hbm, rpb_hbm, tab_v, idx_v, out_v):
    npw = (N * N) // _NW
    wid = lax.axis_index("s") * 2 + lax.axis_index("c")
    base = wid * npw
    pltpu.sync_copy(table_hbm, tab_v)
    pltpu.sync_copy(rpi_hbm.at[pl.ds(base, npw)], idx_v)

    def jbody(j, carry):
        off = pl.multiple_of(16 * j, 16)
        idx = idx_v[pl.ds(off, 16)]
        for h in range(H):
            out_v[h, pl.ds(off, 16)] = plsc.load_gather(
                tab_v, [idx + (h * TABLE)])
        return carry

    lax.fori_loop(0, npw // 16, jbody, 0)
    for h in range(H):
        pltpu.sync_copy(out_v.at[h], rpb_hbm.at[h, pl.ds(base, npw)])


def _rpb_stage(table_t, rpi_flat):
    npw = (N * N) // _NW
    mesh = plsc.VectorSubcoreMesh(core_axis_name="c", subcore_axis_name="s")
    f = pl.kernel(
        _rpb_body,
        out_type=jax.ShapeDtypeStruct((H, N * N), jnp.float32),
        mesh=mesh,
        scratch_types=[
            pltpu.VMEM((_TABPAD,), jnp.float32),
            pltpu.VMEM((npw,), jnp.int32),
            pltpu.VMEM((H, npw), jnp.float32),
        ],
        compiler_params=pltpu.CompilerParams(
            use_tc_tiling_on_sc=False, needs_layout_passes=False),
    )
    return f(table_t, rpi_flat)


def _attn_body(x_ref, rpb_ref, attn_ref):
    x = x_ref[0]
    for h in range(H):
        q = x[:, h * HD:(h + 1) * HD] * SCALE
        k = x[:, DIM + h * HD:DIM + (h + 1) * HD]
        s = lax.dot_general(q, k, (((1,), (1,)), ((), ())),
                            preferred_element_type=jnp.float32)
        s = s + rpb_ref[h]
        m = jnp.max(s, axis=-1, keepdims=True)
        e = jnp.exp(s - m)
        attn_ref[0, h] = e / jnp.sum(e, axis=-1, keepdims=True)


def _attn_stage(qkvp, rpb):
    return pl.pallas_call(
        _attn_body,
        grid=(B,),
        in_specs=[
            pl.BlockSpec((1, N, 4 * DIM), lambda b: (b, 0, 0)),
            pl.BlockSpec((H, N, N), lambda b: (0, 0, 0)),
        ],
        out_specs=pl.BlockSpec((1, H, N, N), lambda b: (b, 0, 0, 0)),
        out_shape=jax.ShapeDtypeStruct((B, H, N, N), jnp.float32),
    )(qkvp, rpb)


_R = B * H * N
_RPW = _R // _NW
_CHUNK = 32
_NCHUNK = _RPW // _CHUNK


def _vsortd(kv):
    return plsc.sort_key_val(kv[0], kv[1], descending=True)


def _vrev(kv):
    return lax.rev(kv[0], (0,)), lax.rev(kv[1], (0,))


def _cleaner(a, b):
    ak, av = a
    bk, bv = b
    c = ak >= bk
    hi = (jnp.where(c, ak, bk), jnp.where(c, av, bv))
    lo = (jnp.where(c, bk, ak), jnp.where(c, bv, av))
    return hi, lo


def _cleaner_hi(a, b):
    ak, av = a
    bk, bv = b
    c = ak >= bk
    return jnp.where(c, ak, bk), jnp.where(c, av, bv)


def _merge2(a, b):
    hi, lo = _cleaner(a, _vrev(b))
    return [_vsortd(hi), _vsortd(lo)]


def _merge4(a, b):
    rb = [_vrev(b[1]), _vrev(b[0])]
    h0, l0 = _cleaner(a[0], rb[0])
    h1, l1 = _cleaner(a[1], rb[1])
    th, tl = _cleaner(h0, h1)
    bh, bl = _cleaner(l0, l1)
    return [_vsortd(th), _vsortd(tl), _vsortd(bh), _vsortd(bl)]


def _merge8_top(a, b):
    rb = [_vrev(b[3]), _vrev(b[2]), _vrev(b[1]), _vrev(b[0])]
    hi = [_cleaner_hi(a[i], rb[i]) for i in range(4)]
    a0, a2 = _cleaner(hi[0], hi[2])
    a1, a3 = _cleaner(hi[1], hi[3])
    b0, b1 = _cleaner(a0, a1)
    b2, b3 = _cleaner(a2, a3)
    return [_vsortd(b0), _vsortd(b1), _vsortd(b2), _vsortd(b3)]


def _row_top64(load16):
    runs16 = [load16(j) for j in range(16)]
    runs32 = [_merge2(runs16[2 * i], runs16[2 * i + 1]) for i in range(8)]
    runs64 = [_merge4(runs32[2 * i], runs32[2 * i + 1]) for i in range(4)]
    t0 = _merge8_top(runs64[0], runs64[1])
    t1 = _merge8_top(runs64[2], runs64[3])
    return _merge8_top(t0, t1)


def _topk_body(attn_hbm, tv_hbm, ti_hbm, bufs, tvb, tib, isems, osems):
    wid = lax.axis_index("s") * 2 + lax.axis_index("c")
    base = wid * _RPW
    iota = lax.iota(jnp.int32, 16)
    idxc = [iota + 16 * j for j in range(16)]

    def in_copy(c, par):
        return pltpu.make_async_copy(
            attn_hbm.at[pl.ds(base + c * _CHUNK, _CHUNK)], bufs.at[par],
            isems.at[par])

    def out_copy_v(c, par):
        return pltpu.make_async_copy(
            tvb.at[par], tv_hbm.at[pl.ds(base + c * _CHUNK, _CHUNK)],
            osems.at[par])

    def out_copy_i(c, par):
        return pltpu.make_async_copy(
            tib.at[par], ti_hbm.at[pl.ds(base + c * _CHUNK, _CHUNK)],
            osems.at[par])

    in_copy(0, 0).start()
    in_copy(1, 1).start()

    def pair_body(c2, carry):
        for par in range(2):
            c = 2 * c2 + par
            in_copy(c, par).wait()

            @pl.when(c >= 2)
            def _():
                out_copy_v(c - 2, par).wait()
                out_copy_i(c - 2, par).wait()

            def row_body(i, carry2):
                def load16(j):
                    return _vsortd((bufs[par, i, pl.ds(16 * j, 16)], idxc[j]))

                t = _row_top64(load16)
                for k in range(4):
                    tvb[par, i, pl.ds(16 * k, 16)] = t[k][0]
                    tib[par, i, pl.ds(16 * k, 16)] = t[k][1]
                return carry2

            lax.fori_loop(0, _CHUNK, row_body, 0)

            @pl.when(c + 2 < _NCHUNK)
            def _():
                in_copy(c + 2, par).start()

            out_copy_v(c, par).start()
            out_copy_i(c, par).start()
        return carry

    lax.fori_loop(0, _NCHUNK // 2, pair_body, 0)
    for par in range(2):
        out_copy_v(_NCHUNK - 2 + par, par).wait()
        out_copy_i(_NCHUNK - 2 + par, par).wait()


def _topk_stage(attn_rows):
    mesh = plsc.VectorSubcoreMesh(core_axis_name="c", subcore_axis_name="s")
    f = pl.kernel(
        _topk_body,
        out_type=(
            jax.ShapeDtypeStruct((_R, TOPK), jnp.float32),
            jax.ShapeDtypeStruct((_R, TOPK), jnp.int32),
        ),
        mesh=mesh,
        scratch_types=[
            pltpu.VMEM((2, _CHUNK, N), jnp.float32),
            pltpu.VMEM((2, _CHUNK, TOPK), jnp.float32),
            pltpu.VMEM((2, _CHUNK, TOPK), jnp.int32),
            pltpu.SemaphoreType.DMA((2,)),
            pltpu.SemaphoreType.DMA((2,)),
        ],
        compiler_params=pltpu.CompilerParams(
            use_tc_tiling_on_sc=True, needs_layout_passes=False),
    )
    return f(attn_rows)


def _out_body(attn_ref, tv_ref, x_ref, w_ref, bias_ref, out_ref):
    x = x_ref[0]
    parts = []
    for h in range(H):
        a = attn_ref[0, h]
        thr = tv_ref[0, h][:, TOPK - 1:TOPK]
        am = jnp.where(a >= thr, a, 0.0)
        v = x[:, 2 * DIM + h * HD:2 * DIM + (h + 1) * HD]
        lepe = x[:, 3 * DIM + h * HD:3 * DIM + (h + 1) * HD]
        parts.append(lax.dot_general(am, v, (((1,), (0,)), ((), ())),
                                     preferred_element_type=jnp.float32) + lepe)
    xo = jnp.concatenate(parts, axis=-1)
    out = lax.dot_general(xo, w_ref[...], (((1,), (1,)), ((), ())),
                          preferred_element_type=jnp.float32)
    out_ref[0] = out + bias_ref[...]


def _out_stage(attn, topv, qkvp, proj_w, proj_b):
    return pl.pallas_call(
        _out_body,
        grid=(B,),
        in_specs=[
            pl.BlockSpec((1, H, N, N), lambda b: (b, 0, 0, 0)),
            pl.BlockSpec((1, H, N, TOPK), lambda b: (b, 0, 0, 0)),
            pl.BlockSpec((1, N, 4 * DIM), lambda b: (b, 0, 0)),
            pl.BlockSpec((DIM, DIM), lambda b: (0, 0)),
            pl.BlockSpec((1, DIM), lambda b: (0, 0)),
        ],
        out_specs=pl.BlockSpec((1, N, DIM), lambda b: (b, 0, 0)),
        out_shape=jax.ShapeDtypeStruct((B, N, DIM), jnp.float32),
    )(attn, topv, qkvp, proj_w, proj_b)


def kernel(qkvp, table, proj_w, proj_b, rpi):
    table_flat = jnp.pad(table.T.reshape(-1), (0, _TABPAD - H * TABLE))
    rpb = _rpb_stage(table_flat, rpi.reshape(-1).astype(jnp.int32))
    rpb = rpb.reshape(H, N, N)

    attn = _attn_stage(qkvp, rpb)

    tv, ti = _topk_stage(attn.reshape(_R, N))
    topv = tv.reshape(B, H, N, TOPK)
    topi = ti.reshape(B, H, N, TOPK)

    xo = _out_stage(attn, topv, qkvp, proj_w, proj_b.reshape(1, DIM))
    return xo, topv, topi


# speedup vs baseline: 1.7021x; 1.1280x over previous
import functools

import jax
import jax.numpy as jnp
from jax import lax
from jax.experimental import pallas as pl
from jax.experimental.pallas import tpu as pltpu
from jax.experimental.pallas import tpu_sc as plsc

B = 64
N = 256
DIM = 192
H = 6
HD = DIM // H
TOPK = 64
TABLE = (2 * 16 - 1) * (2 * 16 - 1)
SCALE = HD ** -0.5


_NW = 32
_TABPAD = 5792


def _rpb_body(table_hbm, rpi_hbm, rpb_hbm, tab_v, idx_v, out_v):
    npw = (N * N) // _NW
    wid = lax.axis_index("s") * 2 + lax.axis_index("c")
    base = wid * npw
    pltpu.sync_copy(table_hbm, tab_v)
    pltpu.sync_copy(rpi_hbm.at[pl.ds(base, npw)], idx_v)

    def jbody(j, carry):
        off = pl.multiple_of(16 * j, 16)
        idx = idx_v[pl.ds(off, 16)]
        for h in range(H):
            out_v[h, pl.ds(off, 16)] = plsc.load_gather(
                tab_v, [idx + (h * TABLE)])
        return carry

    lax.fori_loop(0, npw // 16, jbody, 0)
    for h in range(H):
        pltpu.sync_copy(out_v.at[h], rpb_hbm.at[h, pl.ds(base, npw)])


def _rpb_stage(table_t, rpi_flat):
    npw = (N * N) // _NW
    mesh = plsc.VectorSubcoreMesh(core_axis_name="c", subcore_axis_name="s")
    f = pl.kernel(
        _rpb_body,
        out_type=jax.ShapeDtypeStruct((H, N * N), jnp.float32),
        mesh=mesh,
        scratch_types=[
            pltpu.VMEM((_TABPAD,), jnp.float32),
            pltpu.VMEM((npw,), jnp.int32),
            pltpu.VMEM((H, npw), jnp.float32),
        ],
        compiler_params=pltpu.CompilerParams(
            use_tc_tiling_on_sc=False, needs_layout_passes=False),
    )
    return f(table_t, rpi_flat)


def _attn_body(x_ref, rpb_ref, attn_ref):
    x = x_ref[0]
    for h in range(H):
        q = x[:, h * HD:(h + 1) * HD] * SCALE
        k = x[:, DIM + h * HD:DIM + (h + 1) * HD]
        s = lax.dot_general(q, k, (((1,), (1,)), ((), ())),
                            preferred_element_type=jnp.float32)
        s = s + rpb_ref[h]
        m = jnp.max(s, axis=-1, keepdims=True)
        e = jnp.exp(s - m)
        attn_ref[0, h] = e / jnp.sum(e, axis=-1, keepdims=True)


def _attn_stage(qkvp, rpb, b0, nb):
    return pl.pallas_call(
        _attn_body,
        grid=(nb,),
        in_specs=[
            pl.BlockSpec((1, N, 4 * DIM), lambda b: (b + b0, 0, 0)),
            pl.BlockSpec((H, N, N), lambda b: (0, 0, 0)),
        ],
        out_specs=pl.BlockSpec((1, H, N, N), lambda b: (b, 0, 0, 0)),
        out_shape=jax.ShapeDtypeStruct((nb, H, N, N), jnp.float32),
    )(qkvp, rpb)


_CHUNK = 32
_NCHIP = 4


def _vsortd(kv):
    return plsc.sort_key_val(kv[0], kv[1], descending=True)


def _vrev(kv):
    return lax.rev(kv[0], (0,)), lax.rev(kv[1], (0,))


def _cleaner(a, b):
    ak, av = a
    bk, bv = b
    c = ak >= bk
    hi = (jnp.where(c, ak, bk), jnp.where(c, av, bv))
    lo = (jnp.where(c, bk, ak), jnp.where(c, bv, av))
    return hi, lo


def _cleaner_hi(a, b):
    ak, av = a
    bk, bv = b
    c = ak >= bk
    return jnp.where(c, ak, bk), jnp.where(c, av, bv)


def _merge2(a, b):
    hi, lo = _cleaner(a, _vrev(b))
    return [_vsortd(hi), _vsortd(lo)]


def _merge4(a, b):
    rb = [_vrev(b[1]), _vrev(b[0])]
    h0, l0 = _cleaner(a[0], rb[0])
    h1, l1 = _cleaner(a[1], rb[1])
    th, tl = _cleaner(h0, h1)
    bh, bl = _cleaner(l0, l1)
    return [_vsortd(th), _vsortd(tl), _vsortd(bh), _vsortd(bl)]


def _merge8_top(a, b):
    rb = [_vrev(b[3]), _vrev(b[2]), _vrev(b[1]), _vrev(b[0])]
    hi = [_cleaner_hi(a[i], rb[i]) for i in range(4)]
    a0, a2 = _cleaner(hi[0], hi[2])
    a1, a3 = _cleaner(hi[1], hi[3])
    b0, b1 = _cleaner(a0, a1)
    b2, b3 = _cleaner(a2, a3)
    return [_vsortd(b0), _vsortd(b1), _vsortd(b2), _vsortd(b3)]


def _row_top64(load16):
    runs16 = [load16(j) for j in range(16)]
    runs32 = [_merge2(runs16[2 * i], runs16[2 * i + 1]) for i in range(8)]
    runs64 = [_merge4(runs32[2 * i], runs32[2 * i + 1]) for i in range(4)]
    t0 = _merge8_top(runs64[0], runs64[1])
    t1 = _merge8_top(runs64[2], runs64[3])
    return _merge8_top(t0, t1)


def _topk_body(attn_hbm, tv_hbm, ti_hbm, bufs, tvb, tib, isems, osems,
               *, rpw, nchunk):
    wid = lax.axis_index("s") * 2 + lax.axis_index("c")
    base = wid * rpw
    iota = lax.iota(jnp.int32, 16)
    idxc = [iota + 16 * j for j in range(16)]

    def in_copy(c, par):
        return pltpu.make_async_copy(
            attn_hbm.at[pl.ds(base + c * _CHUNK, _CHUNK)], bufs.at[par],
            isems.at[par])

    def out_copy_v(c, par):
        return pltpu.make_async_copy(
            tvb.at[par], tv_hbm.at[pl.ds(base + c * _CHUNK, _CHUNK)],
            osems.at[par])

    def out_copy_i(c, par):
        return pltpu.make_async_copy(
            tib.at[par], ti_hbm.at[pl.ds(base + c * _CHUNK, _CHUNK)],
            osems.at[par])

    in_copy(0, 0).start()
    in_copy(1, 1).start()

    def pair_body(c2, carry):
        for par in range(2):
            c = 2 * c2 + par
            in_copy(c, par).wait()

            @pl.when(c >= 2)
            def _():
                out_copy_v(c - 2, par).wait()
                out_copy_i(c - 2, par).wait()

            def row_body(i, carry2):
                def load16(j):
                    return _vsortd((bufs[par, i, pl.ds(16 * j, 16)], idxc[j]))

                t = _row_top64(load16)
                for k in range(4):
                    tvb[par, i, pl.ds(16 * k, 16)] = t[k][0]
                    tib[par, i, pl.ds(16 * k, 16)] = t[k][1]
                return carry2

            lax.fori_loop(0, _CHUNK, row_body, 0)

            @pl.when(c + 2 < nchunk)
            def _():
                in_copy(c + 2, par).start()

            out_copy_v(c, par).start()
            out_copy_i(c, par).start()
        return carry

    lax.fori_loop(0, nchunk // 2, pair_body, 0)
    for par in range(2):
        out_copy_v(nchunk - 2 + par, par).wait()
        out_copy_i(nchunk - 2 + par, par).wait()


def _topk_stage(attn_rows):
    rows = attn_rows.shape[0]
    rpw = rows // _NW
    mesh = plsc.VectorSubcoreMesh(core_axis_name="c", subcore_axis_name="s")
    f = pl.kernel(
        functools.partial(_topk_body, rpw=rpw, nchunk=rpw // _CHUNK),
        out_type=(
            jax.ShapeDtypeStruct((rows, TOPK), jnp.float32),
            jax.ShapeDtypeStruct((rows, TOPK), jnp.int32),
        ),
        mesh=mesh,
        scratch_types=[
            pltpu.VMEM((2, _CHUNK, N), jnp.float32),
            pltpu.VMEM((2, _CHUNK, TOPK), jnp.float32),
            pltpu.VMEM((2, _CHUNK, TOPK), jnp.int32),
            pltpu.SemaphoreType.DMA((2,)),
            pltpu.SemaphoreType.DMA((2,)),
        ],
        compiler_params=pltpu.CompilerParams(
            use_tc_tiling_on_sc=True, needs_layout_passes=False),
    )
    return f(attn_rows)


def _out_body(attn_ref, tv_ref, x_ref, w_ref, bias_ref, out_ref):
    x = x_ref[0]
    parts = []
    for h in range(H):
        a = attn_ref[0, h]
        thr = tv_ref[0, h][:, TOPK - 1:TOPK]
        am = jnp.where(a >= thr, a, 0.0)
        v = x[:, 2 * DIM + h * HD:2 * DIM + (h + 1) * HD]
        lepe = x[:, 3 * DIM + h * HD:3 * DIM + (h + 1) * HD]
        parts.append(lax.dot_general(am, v, (((1,), (0,)), ((), ())),
                                     preferred_element_type=jnp.float32) + lepe)
    xo = jnp.concatenate(parts, axis=-1)
    out = lax.dot_general(xo, w_ref[...], (((1,), (1,)), ((), ())),
                          preferred_element_type=jnp.float32)
    out_ref[0] = out + bias_ref[...]


def _out_stage(attn, topv, qkvp, proj_w, proj_b, b0, nb):
    return pl.pallas_call(
        _out_body,
        grid=(nb,),
        in_specs=[
            pl.BlockSpec((1, H, N, N), lambda b: (b, 0, 0, 0)),
            pl.BlockSpec((1, H, N, TOPK), lambda b: (b, 0, 0, 0)),
            pl.BlockSpec((1, N, 4 * DIM), lambda b: (b + b0, 0, 0)),
            pl.BlockSpec((DIM, DIM), lambda b: (0, 0)),
            pl.BlockSpec((1, DIM), lambda b: (0, 0)),
        ],
        out_specs=pl.BlockSpec((1, N, DIM), lambda b: (b, 0, 0)),
        out_shape=jax.ShapeDtypeStruct((nb, N, DIM), jnp.float32),
    )(attn, topv, qkvp, proj_w, proj_b)


def kernel(qkvp, table, proj_w, proj_b, rpi):
    table_flat = jnp.pad(table.T.reshape(-1), (0, _TABPAD - H * TABLE))
    rpb = _rpb_stage(table_flat, rpi.reshape(-1).astype(jnp.int32))
    rpb = rpb.reshape(H, N, N)

    nb = B // _NCHIP
    bias2d = proj_b.reshape(1, DIM)
    xs, tvs, tis = [], [], []
    for c in range(_NCHIP):
        attn = _attn_stage(qkvp, rpb, c * nb, nb)
        tv, ti = _topk_stage(attn.reshape(nb * H * N, N))
        topv = tv.reshape(nb, H, N, TOPK)
        xs.append(_out_stage(attn, topv, qkvp, proj_w, bias2d, c * nb, nb))
        tvs.append(topv)
        tis.append(ti.reshape(nb, H, N, TOPK))
    return (jnp.concatenate(xs), jnp.concatenate(tvs), jnp.concatenate(tis))
